# Initial kernel scaffold; baseline (speedup 1.0000x reference)
import jax
import jax.numpy as jnp
from jax.experimental import pallas as pl


def kernel(node_hidden, edge_index, edge_hidden, node_id, edge_id, W1, b1, W2, b2, ln_gamma, ln_beta):
    raise NotImplementedError("write your pallas kernel here")


# speedup vs baseline: 4.0886x; 4.0886x over previous
import functools

import jax
import jax.numpy as jnp
from jax import lax
from jax.experimental import pallas as pl
from jax.experimental.pallas import tpu as pltpu
from jax.experimental.pallas import tpu_sc as plsc

N = 10000
E = 320000
D = 128
NG = 512

NC = 2
NS = 16
NW = NC * NS
W = 128
NBLK = E // W
ZROWS = 80
NZCHUNK = N // ZROWS

BN = 1000
NB = N // BN


def _sc_aggregate(node_hidden, src, dst, edge_hidden):
    mesh = plsc.VectorSubcoreMesh(core_axis_name="c", subcore_axis_name="s")

    @functools.partial(
        pl.kernel,
        mesh=mesh,
        out_type=jax.ShapeDtypeStruct((NC * N, D), jnp.float32),
        scratch_types=[
            pltpu.VMEM((W,), jnp.int32),
            pltpu.VMEM((W,), jnp.int32),
            pltpu.VMEM((W, D), jnp.float32),
            pltpu.VMEM((W, D), jnp.float32),
            pltpu.VMEM_SHARED((N, D), jnp.float32),
        ],
    )
    def k(nh_hbm, src_hbm, dst_hbm, eh_hbm, out_hbm, sbuf, dbuf, gbuf, ebuf, acc):
        c = lax.axis_index("c")
        s = lax.axis_index("s")
        wid = s * NC + c

        zero = jnp.zeros((16,), jnp.float32)

        @pl.loop(0, ZROWS)
        def _(r):
            for cc in range(0, D, 16):
                gbuf[r, pl.ds(cc, 16)] = zero

        @pl.loop(s, NZCHUNK, step=NS)
        def _(z):
            pltpu.sync_copy(gbuf.at[pl.ds(0, ZROWS)],
                            acc.at[pl.ds(z * ZROWS, ZROWS)])
        plsc.subcore_barrier()

        @pl.loop(wid, NBLK, step=NW)
        def _(blk):
            e0 = blk * W
            pltpu.sync_copy(src_hbm.at[pl.ds(e0, W)], sbuf)
            pltpu.sync_copy(dst_hbm.at[pl.ds(e0, W)], dbuf)
            pltpu.sync_copy(nh_hbm.at[sbuf], gbuf)
            pltpu.sync_copy(eh_hbm.at[pl.ds(e0, W)], ebuf)

            @pl.loop(0, W)
            def _(r):
                for cc in range(0, D, 16):
                    gbuf[r, pl.ds(cc, 16)] = jnp.maximum(
                        gbuf[r, pl.ds(cc, 16)] + ebuf[r, pl.ds(cc, 16)], 0.0)

            pltpu.sync_copy(gbuf, acc.at[dbuf], add=True)

        plsc.subcore_barrier()

        @pl.loop(s, NZCHUNK, step=NS)
        def _(z):
            pltpu.sync_copy(acc.at[pl.ds(z * ZROWS, ZROWS)],
                            out_hbm.at[pl.ds(c * N + z * ZROWS, ZROWS)])

    return k(node_hidden, src, dst, edge_hidden)


def _tc_counts(node_id):
    def body(nid_ref, out_ref):
        @pl.when(pl.program_id(0) == 0)
        def _():
            out_ref[...] = jnp.zeros_like(out_ref)
        chunk = nid_ref[0, 0]
        oh = (chunk[:, None] ==
              lax.broadcasted_iota(jnp.int32, (BN, NG), 1)).astype(jnp.float32)
        out_ref[...] += jnp.sum(oh, axis=0, keepdims=True)

    return pl.pallas_call(
        body,
        grid=(NB,),
        in_specs=[pl.BlockSpec((1, 1, BN), lambda i: (i, 0, 0))],
        out_specs=pl.BlockSpec((1, NG), lambda i: (0, 0)),
        out_shape=jax.ShapeDtypeStruct((1, NG), jnp.float32),
    )(node_id.reshape(NB, 1, BN))


def _tc_dense(node_hidden, p0, p1, node_id3, counts, W1, b1, W2, b2, g, bt):
    def body(nh_ref, p0_ref, p1_ref, nid_ref, cnt_ref,
             W1_ref, b1_ref, W2_ref, b2_ref, g_ref, bt_ref, out_ref):
        x = nh_ref[...]
        h = x + p0_ref[...] + p1_ref[...]
        a = jnp.maximum(
            jnp.dot(h, W1_ref[...], preferred_element_type=jnp.float32)
            + b1_ref[...], 0.0)
        o = jnp.dot(a, W2_ref[...], preferred_element_type=jnp.float32) + b2_ref[...]
        mean = jnp.mean(o, axis=-1, keepdims=True)
        cen = o - mean
        var = jnp.mean(cen * cen, axis=-1, keepdims=True)
        o = cen * lax.rsqrt(var + 1e-5) * g_ref[...] + bt_ref[...]
        nid = nid_ref[0, 0]
        oh = (nid[:, None] ==
              lax.broadcasted_iota(jnp.int32, (BN, NG), 1)).astype(jnp.float32)
        gcnt = jnp.sum(oh * cnt_ref[...], axis=1, keepdims=True)
        o = jnp.maximum(o * lax.rsqrt(gcnt), 0.0)
        out_ref[...] = o + x

    return pl.pallas_call(
        body,
        grid=(NB,),
        in_specs=[
            pl.BlockSpec((BN, D), lambda i: (i, 0)),
            pl.BlockSpec((BN, D), lambda i: (i, 0)),
            pl.BlockSpec((BN, D), lambda i: (i, 0)),
            pl.BlockSpec((1, 1, BN), lambda i: (i, 0, 0)),
            pl.BlockSpec((1, NG), lambda i: (0, 0)),
            pl.BlockSpec((D, 2 * D), lambda i: (0, 0)),
            pl.BlockSpec((1, 2 * D), lambda i: (0, 0)),
            pl.BlockSpec((2 * D, D), lambda i: (0, 0)),
            pl.BlockSpec((1, D), lambda i: (0, 0)),
            pl.BlockSpec((1, D), lambda i: (0, 0)),
            pl.BlockSpec((1, D), lambda i: (0, 0)),
        ],
        out_specs=pl.BlockSpec((BN, D), lambda i: (i, 0)),
        out_shape=jax.ShapeDtypeStruct((N, D), jnp.float32),
    )(node_hidden, p0, p1, node_id3, counts,
      W1, b1.reshape(1, -1), W2, b2.reshape(1, -1),
      g.reshape(1, -1), bt.reshape(1, -1))


def kernel(node_hidden, edge_index, edge_hidden, node_id, edge_id,
           W1, b1, W2, b2, ln_gamma, ln_beta):
    src = edge_index[0]
    dst = edge_index[1]
    partials = _sc_aggregate(node_hidden, src, dst, edge_hidden)
    counts = _tc_counts(node_id)
    out = _tc_dense(node_hidden, partials[:N], partials[N:],
                    node_id.reshape(NB, 1, BN), counts,
                    W1, b1, W2, b2, ln_gamma, ln_beta)
    return out


# speedup vs baseline: 7.3596x; 1.8000x over previous
import functools

import jax
import jax.numpy as jnp
from jax import lax
from jax.experimental import pallas as pl
from jax.experimental.pallas import tpu as pltpu
from jax.experimental.pallas import tpu_sc as plsc

N = 10000
E = 320000
D = 128
NG = 512

NC = 2
NS = 16
NW = NC * NS
EPW = E // NW
W = 80
NJ = EPW // W
ZROWS = 80
NZCHUNK = N // ZROWS

BN = 1000
NB = N // BN


def _sc_aggregate(node_hidden, src, dst, edge_hidden):
    mesh = plsc.VectorSubcoreMesh(core_axis_name="c", subcore_axis_name="s")

    @functools.partial(
        pl.kernel,
        mesh=mesh,
        out_type=jax.ShapeDtypeStruct((NC * N, D), jnp.float32),
        scratch_types=[
            pltpu.VMEM((W,), jnp.int32),
            pltpu.VMEM((W,), jnp.int32),
            pltpu.VMEM((W,), jnp.int32),
            pltpu.VMEM((W,), jnp.int32),
            pltpu.VMEM((W, D), jnp.float32),
            pltpu.VMEM((W, D), jnp.float32),
            pltpu.VMEM((W, D), jnp.float32),
            pltpu.VMEM((W, D), jnp.float32),
            pltpu.VMEM_SHARED((N, D), jnp.float32),
            pltpu.SemaphoreType.DMA,
            pltpu.SemaphoreType.DMA,
            pltpu.SemaphoreType.DMA,
            pltpu.SemaphoreType.DMA,
            pltpu.SemaphoreType.DMA,
            pltpu.SemaphoreType.DMA,
            pltpu.SemaphoreType.DMA,
            pltpu.SemaphoreType.DMA,
        ],
    )
    def k(nh_hbm, src_hbm, dst_hbm, eh_hbm, out_hbm,
          sbuf0, sbuf1, dbuf0, dbuf1, gbuf0, gbuf1, ebuf0, ebuf1, acc,
          ssem0, ssem1, dsem0, dsem1, gsem0, gsem1, esem0, esem1):
        c = lax.axis_index("c")
        s = lax.axis_index("s")
        wid = s * NC + c
        wbase = wid * EPW

        sbufs = (sbuf0, sbuf1)
        dbufs = (dbuf0, dbuf1)
        gbufs = (gbuf0, gbuf1)
        ebufs = (ebuf0, ebuf1)
        ssems = (ssem0, ssem1)
        dsems = (dsem0, dsem1)
        gsems = (gsem0, gsem1)
        esems = (esem0, esem1)

        zero = jnp.zeros((16,), jnp.float32)

        @pl.loop(0, ZROWS)
        def _(r):
            for cc in range(0, D, 16):
                gbuf0[r, pl.ds(cc, 16)] = zero

        @pl.loop(s, NZCHUNK, step=NS)
        def _(z):
            pltpu.sync_copy(gbuf0.at[pl.ds(0, ZROWS)],
                            acc.at[pl.ds(z * ZROWS, ZROWS)])
        plsc.subcore_barrier()

        def start_fetch(j, b):
            e0 = wbase + j * W
            pltpu.async_copy(src_hbm.at[pl.ds(e0, W)], sbufs[b], ssems[b])
            pltpu.async_copy(dst_hbm.at[pl.ds(e0, W)], dbufs[b], dsems[b])
            pltpu.async_copy(eh_hbm.at[pl.ds(e0, W)], ebufs[b], esems[b])

        def start_gather(b):
            pltpu.make_async_copy(src_hbm.at[pl.ds(0, W)], sbufs[b],
                                  ssems[b]).wait()
            pltpu.async_copy(nh_hbm.at[sbufs[b]], gbufs[b], gsems[b])

        def process(b, fetch_j=None, gather_next=True):
            if gather_next:
                start_gather(1 - b)
            pltpu.make_async_copy(eh_hbm.at[pl.ds(0, W)], ebufs[b],
                                  esems[b]).wait()
            pltpu.make_async_copy(nh_hbm.at[sbufs[b]], gbufs[b],
                                  gsems[b]).wait()
            gbuf, ebuf = gbufs[b], ebufs[b]

            @pl.loop(0, W, step=2)
            def _(r):
                for rr in range(2):
                    for cc in range(0, D, 16):
                        gbuf[r + rr, pl.ds(cc, 16)] = jnp.maximum(
                            gbuf[r + rr, pl.ds(cc, 16)]
                            + ebuf[r + rr, pl.ds(cc, 16)], 0.0)

            pltpu.make_async_copy(dst_hbm.at[pl.ds(0, W)], dbufs[b],
                                  dsems[b]).wait()
            pltpu.sync_copy(gbuf, acc.at[dbufs[b]], add=True)
            if fetch_j is not None:
                start_fetch(fetch_j, b)

        start_fetch(0, 0)
        start_fetch(1, 1)
        start_gather(0)

        @pl.loop(0, (NJ - 3) // 2)
        def _(t):
            j0 = 2 * t
            process(0, fetch_j=j0 + 2)
            process(1, fetch_j=j0 + 3)

        process(0, fetch_j=NJ - 1)
        process(1, fetch_j=None)
        process(0, fetch_j=None, gather_next=False)
        plsc.subcore_barrier()

        @pl.loop(s, NZCHUNK, step=NS)
        def _(z):
            pltpu.sync_copy(acc.at[pl.ds(z * ZROWS, ZROWS)],
                            out_hbm.at[pl.ds(c * N + z * ZROWS, ZROWS)])

    return k(node_hidden, src, dst, edge_hidden)


def _tc_counts(node_id):
    def body(nid_ref, out_ref):
        @pl.when(pl.program_id(0) == 0)
        def _():
            out_ref[...] = jnp.zeros_like(out_ref)
        chunk = nid_ref[0, 0]
        oh = (chunk[:, None] ==
              lax.broadcasted_iota(jnp.int32, (BN, NG), 1)).astype(jnp.float32)
        out_ref[...] += jnp.sum(oh, axis=0, keepdims=True)

    return pl.pallas_call(
        body,
        grid=(NB,),
        in_specs=[pl.BlockSpec((1, 1, BN), lambda i: (i, 0, 0))],
        out_specs=pl.BlockSpec((1, NG), lambda i: (0, 0)),
        out_shape=jax.ShapeDtypeStruct((1, NG), jnp.float32),
    )(node_id.reshape(NB, 1, BN))


def _tc_dense(node_hidden, p0, p1, node_id3, counts, W1, b1, W2, b2, g, bt):
    def body(nh_ref, p0_ref, p1_ref, nid_ref, cnt_ref,
             W1_ref, b1_ref, W2_ref, b2_ref, g_ref, bt_ref, out_ref):
        x = nh_ref[...]
        h = x + p0_ref[...] + p1_ref[...]
        a = jnp.maximum(
            jnp.dot(h, W1_ref[...], preferred_element_type=jnp.float32)
            + b1_ref[...], 0.0)
        o = jnp.dot(a, W2_ref[...], preferred_element_type=jnp.float32) + b2_ref[...]
        mean = jnp.mean(o, axis=-1, keepdims=True)
        cen = o - mean
        var = jnp.mean(cen * cen, axis=-1, keepdims=True)
        o = cen * lax.rsqrt(var + 1e-5) * g_ref[...] + bt_ref[...]
        nid = nid_ref[0, 0]
        oh = (nid[:, None] ==
              lax.broadcasted_iota(jnp.int32, (BN, NG), 1)).astype(jnp.float32)
        gcnt = jnp.sum(oh * cnt_ref[...], axis=1, keepdims=True)
        o = jnp.maximum(o * lax.rsqrt(gcnt), 0.0)
        out_ref[...] = o + x

    return pl.pallas_call(
        body,
        grid=(NB,),
        in_specs=[
            pl.BlockSpec((BN, D), lambda i: (i, 0)),
            pl.BlockSpec((BN, D), lambda i: (i, 0)),
            pl.BlockSpec((BN, D), lambda i: (i, 0)),
            pl.BlockSpec((1, 1, BN), lambda i: (i, 0, 0)),
            pl.BlockSpec((1, NG), lambda i: (0, 0)),
            pl.BlockSpec((D, 2 * D), lambda i: (0, 0)),
            pl.BlockSpec((1, 2 * D), lambda i: (0, 0)),
            pl.BlockSpec((2 * D, D), lambda i: (0, 0)),
            pl.BlockSpec((1, D), lambda i: (0, 0)),
            pl.BlockSpec((1, D), lambda i: (0, 0)),
            pl.BlockSpec((1, D), lambda i: (0, 0)),
        ],
        out_specs=pl.BlockSpec((BN, D), lambda i: (i, 0)),
        out_shape=jax.ShapeDtypeStruct((N, D), jnp.float32),
    )(node_hidden, p0, p1, node_id3, counts,
      W1, b1.reshape(1, -1), W2, b2.reshape(1, -1),
      g.reshape(1, -1), bt.reshape(1, -1))


def kernel(node_hidden, edge_index, edge_hidden, node_id, edge_id,
           W1, b1, W2, b2, ln_gamma, ln_beta):
    src = edge_index[0]
    dst = edge_index[1]
    partials = _sc_aggregate(node_hidden, src, dst, edge_hidden)
    counts = _tc_counts(node_id)
    out = _tc_dense(node_hidden, partials[:N], partials[N:],
                    node_id.reshape(NB, 1, BN), counts,
                    W1, b1, W2, b2, ln_gamma, ln_beta)
    return out
